# baseline (speedup 1.0000x reference)
import functools

import jax
import jax.numpy as jnp
from jax import lax
from jax.experimental import pallas as pl
from jax.experimental.pallas import tpu as pltpu
from jax.experimental.pallas import tpu_sc as plsc

LATENT = 32
HALF = LATENT // 2
ROWS = 1000000
ROWS_D = 999936
TAIL_N = ROWS - ROWS_D
BATCH = 16384
NC, NS, L = 2, 16, 16
NW = NC * NS
B_PER_W = BATCH // NW
BLK = 128
QN = B_PER_W // BLK
G_PER_BLK = BLK // L

CW = 32256
DT_N = ROWS_D // CW


def _detile_body(in_ref, out_hbm, pk, sem):
    k = pl.program_id(0)
    rnd = jnp.uint32(0x8000)
    lo32 = lax.bitcast_convert_type(in_ref[pl.ds(0, HALF), :], jnp.uint32) \
        + rnd
    hi32 = lax.bitcast_convert_type(in_ref[pl.ds(HALF, HALF), :], jnp.uint32) \
        + rnd
    pk[:, :] = lax.bitwise_or(
        lax.shift_right_logical(lo32, jnp.uint32(16)),
        lax.bitwise_and(hi32, jnp.uint32(0xFFFF0000)))
    cps = []
    for dp in range(HALF):
        cps.append(pltpu.make_async_copy(
            pk.at[dp],
            out_hbm.at[pl.ds(dp * ROWS_D + k * CW, CW)],
            sem))
    for c in cps:
        c.start()
    for c in cps:
        c.wait()


def _detile(tT):
    return pl.pallas_call(
        _detile_body,
        grid=(DT_N,),
        in_specs=[pl.BlockSpec((LATENT, CW), lambda k: (0, k))],
        out_specs=pl.BlockSpec(memory_space=pl.ANY),
        out_shape=jax.ShapeDtypeStruct((HALF * ROWS_D,), jnp.uint32),
        scratch_shapes=[
            pltpu.VMEM((HALF, CW), jnp.uint32),
            pltpu.SemaphoreType.DMA,
        ],
    )(tT)


def _mf_body(uid_hbm, iid_hbm, ut_hbm, it_hbm, utail_hbm, itail_hbm, out_hbm,
             uid_v, iid_v, uidx, iidx, uval, ival, utail_v, itail_v,
             out_v, sem):
    wid = lax.axis_index("s") * NC + lax.axis_index("c")
    base = wid * B_PER_W

    pltpu.sync_copy(uid_hbm.at[pl.ds(base, B_PER_W)], uid_v)
    pltpu.sync_copy(iid_hbm.at[pl.ds(base, B_PER_W)], iid_v)
    pltpu.sync_copy(utail_hbm, utail_v)
    pltpu.sync_copy(itail_hbm, itail_v)

    rmax = jnp.full((L,), ROWS_D - 1, jnp.int32)

    @pl.loop(0, QN)
    def _build(q):
        for g in range(G_PER_BLK):
            uvec = jnp.minimum(uid_v[pl.ds(q * BLK + g * L, L)], rmax)
            ivec = jnp.minimum(iid_v[pl.ds(q * BLK + g * L, L)], rmax)
            for dp in range(HALF):
                off = jnp.full((L,), dp * ROWS_D, jnp.int32)
                uidx[dp * QN + q, pl.ds(g * L, L)] = uvec + off
                iidx[dp * QN + q, pl.ds(g * L, L)] = ivec + off

    descs = []
    for r in range(HALF * QN):
        descs.append(pltpu.async_copy(
            ut_hbm.at[uidx.at[r]], uval.at[r], sem))
        descs.append(pltpu.async_copy(
            it_hbm.at[iidx.at[r]], ival.at[r], sem))
    for dsc in descs:
        dsc.wait()

    tlo = jnp.full((L,), ROWS_D, jnp.int32)
    himask = jnp.full((L,), 0xFFFF0000, jnp.uint32)

    @pl.loop(0, QN)
    def _compute(q):
        for g in range(G_PER_BLK):
            uvec = uid_v[pl.ds(q * BLK + g * L, L)]
            ivec = iid_v[pl.ds(q * BLK + g * L, L)]
            umask = uvec >= tlo
            imask = ivec >= tlo
            uloc = jnp.where(umask, (uvec - tlo) * LATENT, 0)
            iloc = jnp.where(imask, (ivec - tlo) * LATENT, 0)
            acc = jnp.zeros((L,), jnp.float32)
            for dp in range(HALF):
                uw = uval[dp * QN + q, pl.ds(g * L, L)]
                vw = ival[dp * QN + q, pl.ds(g * L, L)]
                u_lo = plsc.bitcast(lax.shift_left(uw, jnp.uint32(16)), jnp.float32)
                u_hi = plsc.bitcast(lax.bitwise_and(uw, himask), jnp.float32)
                v_lo = plsc.bitcast(lax.shift_left(vw, jnp.uint32(16)), jnp.float32)
                v_hi = plsc.bitcast(lax.bitwise_and(vw, himask), jnp.float32)
                d0 = jnp.full((L,), dp, jnp.int32)
                d1 = jnp.full((L,), dp + HALF, jnp.int32)
                u_lo = jnp.where(
                    umask, plsc.load_gather(utail_v, [uloc + d0]), u_lo)
                u_hi = jnp.where(
                    umask, plsc.load_gather(utail_v, [uloc + d1]), u_hi)
                v_lo = jnp.where(
                    imask, plsc.load_gather(itail_v, [iloc + d0]), v_lo)
                v_hi = jnp.where(
                    imask, plsc.load_gather(itail_v, [iloc + d1]), v_hi)
                acc = acc + u_lo * v_lo + u_hi * v_hi
            out_v[pl.ds(q * BLK + g * L, L)] = acc

    pltpu.sync_copy(out_v, out_hbm.at[pl.ds(base, B_PER_W)])


@jax.jit
def _mf(user_id, item_id, utT, itT, utail, itail):
    ut1 = _detile(utT)
    it1 = _detile(itT)
    mesh = plsc.VectorSubcoreMesh(
        core_axis_name="c", subcore_axis_name="s",
        num_cores=NC, num_subcores=NS)
    run = functools.partial(
        pl.kernel,
        out_type=jax.ShapeDtypeStruct((BATCH,), jnp.float32),
        mesh=mesh,
        compiler_params=pltpu.CompilerParams(needs_layout_passes=False),
        scratch_types=[
            pltpu.VMEM((B_PER_W,), jnp.int32),
            pltpu.VMEM((B_PER_W,), jnp.int32),
            pltpu.VMEM((HALF * QN, BLK), jnp.int32),
            pltpu.VMEM((HALF * QN, BLK), jnp.int32),
            pltpu.VMEM((HALF * QN, BLK), jnp.uint32),
            pltpu.VMEM((HALF * QN, BLK), jnp.uint32),
            pltpu.VMEM((TAIL_N * LATENT,), jnp.float32),
            pltpu.VMEM((TAIL_N * LATENT,), jnp.float32),
            pltpu.VMEM((B_PER_W,), jnp.float32),
            pltpu.SemaphoreType.DMA,
        ],
    )(_mf_body)
    return run(user_id, item_id, ut1, it1, utail, itail)


def kernel(user_id, item_id, user_table, item_table):
    utail = user_table[ROWS_D:].reshape(-1)
    itail = item_table[ROWS_D:].reshape(-1)
    return _mf(user_id.astype(jnp.int32), item_id.astype(jnp.int32),
               user_table.T, item_table.T, utail, itail)


# speedup vs baseline: 1.1066x; 1.1066x over previous
import functools

import jax
import jax.numpy as jnp
from jax import lax
from jax.experimental import pallas as pl
from jax.experimental.pallas import tpu as pltpu
from jax.experimental.pallas import tpu_sc as plsc

LATENT = 32
HALF = LATENT // 2
ROWS = 1000000
ROWS_D = 999936
TAIL_N = ROWS - ROWS_D
BATCH = 16384
NC, NS, L = 2, 16, 16
NW = NC * NS
B_PER_W = BATCH // NW
BLK = 128
QN = B_PER_W // BLK
G_PER_BLK = BLK // L

CW = 32256
DT_N = ROWS_D // CW


def _detile_body(in_ref, out_hbm, pk0, pk1, sem0, sem1):
    k = pl.program_id(0)
    rnd = jnp.uint32(0x8000)
    lo32 = lax.bitcast_convert_type(in_ref[pl.ds(0, HALF), :], jnp.uint32) \
        + rnd
    hi32 = lax.bitcast_convert_type(in_ref[pl.ds(HALF, HALF), :], jnp.uint32) \
        + rnd
    packed = lax.bitwise_or(
        lax.shift_right_logical(lo32, jnp.uint32(16)),
        lax.bitwise_and(hi32, jnp.uint32(0xFFFF0000)))

    def cps(buf, sem, kk):
        return [pltpu.make_async_copy(
            buf.at[dp], out_hbm.at[pl.ds(dp * ROWS_D + kk * CW, CW)], sem)
            for dp in range(HALF)]

    def step(buf, sem):
        @pl.when(k >= 2)
        def _():
            for c in cps(buf, sem, k - 2):
                c.wait()
        buf[:, :] = packed
        for c in cps(buf, sem, k):
            c.start()

    @pl.when(lax.rem(k, 2) == 0)
    def _():
        step(pk0, sem0)

    @pl.when(lax.rem(k, 2) == 1)
    def _():
        step(pk1, sem1)

    @pl.when(k == DT_N - 1)
    def _():
        for c in cps(pk1, sem1, k - 1):
            c.wait()
        for c in cps(pk0, sem0, k):
            c.wait()


def _detile(tT):
    return pl.pallas_call(
        _detile_body,
        grid=(DT_N,),
        in_specs=[pl.BlockSpec((LATENT, CW), lambda k: (0, k))],
        out_specs=pl.BlockSpec(memory_space=pl.ANY),
        out_shape=jax.ShapeDtypeStruct((HALF * ROWS_D,), jnp.uint32),
        scratch_shapes=[
            pltpu.VMEM((HALF, CW), jnp.uint32),
            pltpu.VMEM((HALF, CW), jnp.uint32),
            pltpu.SemaphoreType.DMA,
            pltpu.SemaphoreType.DMA,
        ],
    )(tT)


def _mf_body(uid_hbm, iid_hbm, ut_hbm, it_hbm, utail_hbm, itail_hbm, out_hbm,
             uid_v, iid_v, uidx, iidx, uval, ival, utail_v, itail_v,
             out_v, sem):
    wid = lax.axis_index("s") * NC + lax.axis_index("c")
    base = wid * B_PER_W

    pltpu.sync_copy(uid_hbm.at[pl.ds(base, B_PER_W)], uid_v)
    pltpu.sync_copy(iid_hbm.at[pl.ds(base, B_PER_W)], iid_v)
    pltpu.sync_copy(utail_hbm, utail_v)
    pltpu.sync_copy(itail_hbm, itail_v)

    rmax = jnp.full((L,), ROWS_D - 1, jnp.int32)

    @pl.loop(0, QN)
    def _build(q):
        for g in range(G_PER_BLK):
            uvec = jnp.minimum(uid_v[pl.ds(q * BLK + g * L, L)], rmax)
            ivec = jnp.minimum(iid_v[pl.ds(q * BLK + g * L, L)], rmax)
            for dp in range(HALF):
                off = jnp.full((L,), dp * ROWS_D, jnp.int32)
                uidx[dp * QN + q, pl.ds(g * L, L)] = uvec + off
                iidx[dp * QN + q, pl.ds(g * L, L)] = ivec + off

    descs = []
    for r in range(HALF * QN):
        descs.append(pltpu.async_copy(
            ut_hbm.at[uidx.at[r]], uval.at[r], sem))
        descs.append(pltpu.async_copy(
            it_hbm.at[iidx.at[r]], ival.at[r], sem))
    for dsc in descs:
        dsc.wait()

    tlo = jnp.full((L,), ROWS_D, jnp.int32)
    himask = jnp.full((L,), 0xFFFF0000, jnp.uint32)

    @pl.loop(0, QN)
    def _compute(q):
        for g in range(G_PER_BLK):
            uvec = uid_v[pl.ds(q * BLK + g * L, L)]
            ivec = iid_v[pl.ds(q * BLK + g * L, L)]
            umask = uvec >= tlo
            imask = ivec >= tlo
            uloc = jnp.where(umask, (uvec - tlo) * LATENT, 0)
            iloc = jnp.where(imask, (ivec - tlo) * LATENT, 0)
            acc = jnp.zeros((L,), jnp.float32)
            for dp in range(HALF):
                uw = uval[dp * QN + q, pl.ds(g * L, L)]
                vw = ival[dp * QN + q, pl.ds(g * L, L)]
                u_lo = plsc.bitcast(lax.shift_left(uw, jnp.uint32(16)), jnp.float32)
                u_hi = plsc.bitcast(lax.bitwise_and(uw, himask), jnp.float32)
                v_lo = plsc.bitcast(lax.shift_left(vw, jnp.uint32(16)), jnp.float32)
                v_hi = plsc.bitcast(lax.bitwise_and(vw, himask), jnp.float32)
                d0 = jnp.full((L,), dp, jnp.int32)
                d1 = jnp.full((L,), dp + HALF, jnp.int32)
                u_lo = jnp.where(
                    umask, plsc.load_gather(utail_v, [uloc + d0]), u_lo)
                u_hi = jnp.where(
                    umask, plsc.load_gather(utail_v, [uloc + d1]), u_hi)
                v_lo = jnp.where(
                    imask, plsc.load_gather(itail_v, [iloc + d0]), v_lo)
                v_hi = jnp.where(
                    imask, plsc.load_gather(itail_v, [iloc + d1]), v_hi)
                acc = acc + u_lo * v_lo + u_hi * v_hi
            out_v[pl.ds(q * BLK + g * L, L)] = acc

    pltpu.sync_copy(out_v, out_hbm.at[pl.ds(base, B_PER_W)])


@jax.jit
def _mf(user_id, item_id, utT, itT, utail, itail):
    ut1 = _detile(utT)
    it1 = _detile(itT)
    mesh = plsc.VectorSubcoreMesh(
        core_axis_name="c", subcore_axis_name="s",
        num_cores=NC, num_subcores=NS)
    run = functools.partial(
        pl.kernel,
        out_type=jax.ShapeDtypeStruct((BATCH,), jnp.float32),
        mesh=mesh,
        compiler_params=pltpu.CompilerParams(needs_layout_passes=False),
        scratch_types=[
            pltpu.VMEM((B_PER_W,), jnp.int32),
            pltpu.VMEM((B_PER_W,), jnp.int32),
            pltpu.VMEM((HALF * QN, BLK), jnp.int32),
            pltpu.VMEM((HALF * QN, BLK), jnp.int32),
            pltpu.VMEM((HALF * QN, BLK), jnp.uint32),
            pltpu.VMEM((HALF * QN, BLK), jnp.uint32),
            pltpu.VMEM((TAIL_N * LATENT,), jnp.float32),
            pltpu.VMEM((TAIL_N * LATENT,), jnp.float32),
            pltpu.VMEM((B_PER_W,), jnp.float32),
            pltpu.SemaphoreType.DMA,
        ],
    )(_mf_body)
    return run(user_id, item_id, ut1, it1, utail, itail)


def kernel(user_id, item_id, user_table, item_table):
    utail = user_table[ROWS_D:].reshape(-1)
    itail = item_table[ROWS_D:].reshape(-1)
    return _mf(user_id.astype(jnp.int32), item_id.astype(jnp.int32),
               user_table.T, item_table.T, utail, itail)


# speedup vs baseline: 1.1626x; 1.0506x over previous
import functools

import jax
import jax.numpy as jnp
from jax import lax
from jax.experimental import pallas as pl
from jax.experimental.pallas import tpu as pltpu
from jax.experimental.pallas import tpu_sc as plsc

LATENT = 32
HALF = LATENT // 2
ROWS = 1000000
ROWS_D = 999936
TAIL_N = ROWS - ROWS_D
BATCH = 16384
NC, NS, L = 2, 16, 16
NW = NC * NS
B_PER_W = BATCH // NW
BLK = 128
QN = B_PER_W // BLK
G_PER_BLK = BLK // L

CW = 83328
DT_N = ROWS_D // CW


def _detile_body(in_ref, out_hbm, pk0, pk1, sem0, sem1):
    k = pl.program_id(0)
    rnd = jnp.uint32(0x8000)
    lo32 = lax.bitcast_convert_type(in_ref[pl.ds(0, HALF), :], jnp.uint32) \
        + rnd
    hi32 = lax.bitcast_convert_type(in_ref[pl.ds(HALF, HALF), :], jnp.uint32) \
        + rnd
    packed = lax.bitwise_or(
        lax.shift_right_logical(lo32, jnp.uint32(16)),
        lax.bitwise_and(hi32, jnp.uint32(0xFFFF0000)))

    def cps(buf, sem, kk):
        return [pltpu.make_async_copy(
            buf.at[dp], out_hbm.at[pl.ds(dp * ROWS_D + kk * CW, CW)], sem)
            for dp in range(HALF)]

    def step(buf, sem):
        @pl.when(k >= 2)
        def _():
            for c in cps(buf, sem, k - 2):
                c.wait()
        buf[:, :] = packed
        for c in cps(buf, sem, k):
            c.start()

    @pl.when(lax.rem(k, 2) == 0)
    def _():
        step(pk0, sem0)

    @pl.when(lax.rem(k, 2) == 1)
    def _():
        step(pk1, sem1)

    last, prev = (pk0, pk1) if (DT_N - 1) % 2 == 0 else (pk1, pk0)
    lsem, psem = (sem0, sem1) if (DT_N - 1) % 2 == 0 else (sem1, sem0)

    @pl.when(k == DT_N - 1)
    def _():
        for c in cps(prev, psem, k - 1):
            c.wait()
        for c in cps(last, lsem, k):
            c.wait()


def _detile(tT):
    return pl.pallas_call(
        _detile_body,
        grid=(DT_N,),
        in_specs=[pl.BlockSpec((LATENT, CW), lambda k: (0, k))],
        out_specs=pl.BlockSpec(memory_space=pl.ANY),
        out_shape=jax.ShapeDtypeStruct((HALF * ROWS_D,), jnp.uint32),
        scratch_shapes=[
            pltpu.VMEM((HALF, CW), jnp.uint32),
            pltpu.VMEM((HALF, CW), jnp.uint32),
            pltpu.SemaphoreType.DMA,
            pltpu.SemaphoreType.DMA,
        ],
    )(tT)


def _mf_body(uid_hbm, iid_hbm, ut_hbm, it_hbm, utail_hbm, itail_hbm, out_hbm,
             uid_v, iid_v, uidx, iidx, uval, ival, utail_v, itail_v,
             out_v, sem):
    wid = lax.axis_index("s") * NC + lax.axis_index("c")
    base = wid * B_PER_W

    pltpu.sync_copy(uid_hbm.at[pl.ds(base, B_PER_W)], uid_v)
    pltpu.sync_copy(iid_hbm.at[pl.ds(base, B_PER_W)], iid_v)
    pltpu.sync_copy(utail_hbm, utail_v)
    pltpu.sync_copy(itail_hbm, itail_v)

    rmax = jnp.full((L,), ROWS_D - 1, jnp.int32)

    @pl.loop(0, QN)
    def _build(q):
        for g in range(G_PER_BLK):
            uvec = jnp.minimum(uid_v[pl.ds(q * BLK + g * L, L)], rmax)
            ivec = jnp.minimum(iid_v[pl.ds(q * BLK + g * L, L)], rmax)
            for dp in range(HALF):
                off = jnp.full((L,), dp * ROWS_D, jnp.int32)
                uidx[dp * QN + q, pl.ds(g * L, L)] = uvec + off
                iidx[dp * QN + q, pl.ds(g * L, L)] = ivec + off

    descs = []
    for r in range(HALF * QN):
        descs.append(pltpu.async_copy(
            ut_hbm.at[uidx.at[r]], uval.at[r], sem))
        descs.append(pltpu.async_copy(
            it_hbm.at[iidx.at[r]], ival.at[r], sem))
    for dsc in descs:
        dsc.wait()

    tlo = jnp.full((L,), ROWS_D, jnp.int32)
    himask = jnp.full((L,), 0xFFFF0000, jnp.uint32)

    @pl.loop(0, QN)
    def _compute(q):
        for g in range(G_PER_BLK):
            uvec = uid_v[pl.ds(q * BLK + g * L, L)]
            ivec = iid_v[pl.ds(q * BLK + g * L, L)]
            umask = uvec >= tlo
            imask = ivec >= tlo
            uloc = jnp.where(umask, (uvec - tlo) * LATENT, 0)
            iloc = jnp.where(imask, (ivec - tlo) * LATENT, 0)
            acc = jnp.zeros((L,), jnp.float32)
            for dp in range(HALF):
                uw = uval[dp * QN + q, pl.ds(g * L, L)]
                vw = ival[dp * QN + q, pl.ds(g * L, L)]
                u_lo = plsc.bitcast(lax.shift_left(uw, jnp.uint32(16)), jnp.float32)
                u_hi = plsc.bitcast(lax.bitwise_and(uw, himask), jnp.float32)
                v_lo = plsc.bitcast(lax.shift_left(vw, jnp.uint32(16)), jnp.float32)
                v_hi = plsc.bitcast(lax.bitwise_and(vw, himask), jnp.float32)
                d0 = jnp.full((L,), dp, jnp.int32)
                d1 = jnp.full((L,), dp + HALF, jnp.int32)
                u_lo = jnp.where(
                    umask, plsc.load_gather(utail_v, [uloc + d0]), u_lo)
                u_hi = jnp.where(
                    umask, plsc.load_gather(utail_v, [uloc + d1]), u_hi)
                v_lo = jnp.where(
                    imask, plsc.load_gather(itail_v, [iloc + d0]), v_lo)
                v_hi = jnp.where(
                    imask, plsc.load_gather(itail_v, [iloc + d1]), v_hi)
                acc = acc + u_lo * v_lo + u_hi * v_hi
            out_v[pl.ds(q * BLK + g * L, L)] = acc

    pltpu.sync_copy(out_v, out_hbm.at[pl.ds(base, B_PER_W)])


@jax.jit
def _mf(user_id, item_id, utT, itT, utail, itail):
    ut1 = _detile(utT)
    it1 = _detile(itT)
    mesh = plsc.VectorSubcoreMesh(
        core_axis_name="c", subcore_axis_name="s",
        num_cores=NC, num_subcores=NS)
    run = functools.partial(
        pl.kernel,
        out_type=jax.ShapeDtypeStruct((BATCH,), jnp.float32),
        mesh=mesh,
        compiler_params=pltpu.CompilerParams(needs_layout_passes=False),
        scratch_types=[
            pltpu.VMEM((B_PER_W,), jnp.int32),
            pltpu.VMEM((B_PER_W,), jnp.int32),
            pltpu.VMEM((HALF * QN, BLK), jnp.int32),
            pltpu.VMEM((HALF * QN, BLK), jnp.int32),
            pltpu.VMEM((HALF * QN, BLK), jnp.uint32),
            pltpu.VMEM((HALF * QN, BLK), jnp.uint32),
            pltpu.VMEM((TAIL_N * LATENT,), jnp.float32),
            pltpu.VMEM((TAIL_N * LATENT,), jnp.float32),
            pltpu.VMEM((B_PER_W,), jnp.float32),
            pltpu.SemaphoreType.DMA,
        ],
    )(_mf_body)
    return run(user_id, item_id, ut1, it1, utail, itail)


def kernel(user_id, item_id, user_table, item_table):
    utail = user_table[ROWS_D:].reshape(-1)
    itail = item_table[ROWS_D:].reshape(-1)
    return _mf(user_id.astype(jnp.int32), item_id.astype(jnp.int32),
               user_table.T, item_table.T, utail, itail)
